# baseline (speedup 1.0000x reference)
import functools
import math

import jax
import jax.numpy as jnp
from jax import lax
from jax.experimental import pallas as pl
from jax.experimental.pallas import tpu as pltpu
from jax.experimental.pallas import tpu_sc as plsc

D_MODEL = 64
SCALE = math.sqrt(D_MODEL)

NC, NS = 2, 16
NW = NC * NS
LANES = 16

CHUNK = 128
NBUF = 10
G_AHEAD = 7


@functools.cache
def _build(n_idx):
    assert n_idx % (NW * CHUNK) == 0
    b_per_w = n_idx // NW
    n_steps = b_per_w // CHUNK
    assert n_steps % NBUF == 0 and n_steps > NBUF

    mesh = plsc.VectorSubcoreMesh(core_axis_name="c", subcore_axis_name="s")

    @functools.partial(
        pl.kernel,
        out_type=jax.ShapeDtypeStruct((n_idx, D_MODEL), jnp.float32),
        mesh=mesh,
        scratch_types=[
            pltpu.VMEM((b_per_w,), jnp.int32),
            pltpu.VMEM((NBUF * CHUNK, D_MODEL), jnp.float32),
            pltpu.SemaphoreType.DMA,
            pltpu.SemaphoreType.DMA,
        ],
        compiler_params=pltpu.CompilerParams(use_tc_tiling_on_sc=False),
    )
    def emb_kernel(idx_hbm, lut_hbm, out_hbm, idx_v, rows_v, gsem, wsem):
        wid = lax.axis_index("s") * NC + lax.axis_index("c")
        base = wid * b_per_w

        pltpu.sync_copy(idx_hbm.at[pl.ds(base, b_per_w)], idx_v)

        def buf(j):
            return rows_v.at[pl.ds(j * CHUNK, CHUNK)]

        def fire_gather(s, j):
            pltpu.async_copy(
                lut_hbm.at[idx_v.at[pl.ds(pl.multiple_of(s * CHUNK, CHUNK),
                                          CHUNK)]],
                buf(j), gsem)

        for s in range(G_AHEAD):
            fire_gather(s, s)

        @pl.loop(0, n_steps, step=NBUF)
        def outer(g):
            for b in range(NBUF):
                s = g + b
                pltpu.make_async_copy(
                    lut_hbm.at[pl.ds(0, CHUNK)], buf(b), gsem).wait()

                @plsc.parallel_loop(0, CHUNK, unroll=8)
                def scale_row(i):
                    for c in range(D_MODEL // LANES):
                        sl = (b * CHUNK + i, pl.ds(c * LANES, LANES))
                        rows_v[sl] = rows_v[sl] * SCALE

                pltpu.async_copy(
                    buf(b),
                    out_hbm.at[pl.ds(pl.multiple_of(base + s * CHUNK, CHUNK),
                                     CHUNK)],
                    wsem)

                nb = (b + G_AHEAD) % NBUF

                @pl.when(jnp.logical_and(s >= NBUF - G_AHEAD,
                                         s + G_AHEAD < n_steps))
                def _():
                    pltpu.make_async_copy(
                        buf(nb), out_hbm.at[pl.ds(0, CHUNK)], wsem).wait()

                @pl.when(s + G_AHEAD < n_steps)
                def _():
                    fire_gather(s + G_AHEAD, nb)

        for _ in range(NBUF):
            pltpu.make_async_copy(
                buf(0), out_hbm.at[pl.ds(0, CHUNK)], wsem).wait()

    return emb_kernel


def kernel(x, lut):
    b, s = x.shape
    n = b * s
    idx = x.reshape(n).astype(jnp.int32)
    out = _build(n)(idx, lut)
    return out.reshape(b, s, D_MODEL)


# speedup vs baseline: 1.0041x; 1.0041x over previous
import functools
import math

import jax
import jax.numpy as jnp
from jax import lax
from jax.experimental import pallas as pl
from jax.experimental.pallas import tpu as pltpu
from jax.experimental.pallas import tpu_sc as plsc

D_MODEL = 64
SCALE = math.sqrt(D_MODEL)

NC, NS = 2, 16
NW = NC * NS
LANES = 16

NBUF = 8
G_AHEAD = 5


@functools.cache
def _build(batch, seq):
    assert batch % NW == 0
    b_per_w = batch // NW
    n_steps = b_per_w
    assert n_steps % NBUF == 0 and n_steps > NBUF

    mesh = plsc.VectorSubcoreMesh(core_axis_name="c", subcore_axis_name="s")

    @functools.partial(
        pl.kernel,
        out_type=jax.ShapeDtypeStruct((batch, seq, D_MODEL), jnp.float32),
        mesh=mesh,
        scratch_types=[
            pltpu.VMEM((b_per_w, seq), jnp.int32),
            pltpu.VMEM((NBUF * seq, D_MODEL), jnp.float32),
            pltpu.SemaphoreType.DMA,
            pltpu.SemaphoreType.DMA,
        ],
        compiler_params=pltpu.CompilerParams(use_tc_tiling_on_sc=False),
    )
    def emb_kernel(idx_hbm, lut_hbm, out_hbm, idx_v, rows_v, gsem, wsem):
        wid = lax.axis_index("s") * NC + lax.axis_index("c")
        b0 = wid * b_per_w

        pltpu.sync_copy(idx_hbm.at[pl.ds(b0, b_per_w)], idx_v)

        def buf(j):
            return rows_v.at[pl.ds(j * seq, seq)]

        def fire_gather(s, j):
            pltpu.async_copy(lut_hbm.at[idx_v.at[s]], buf(j), gsem)

        for s in range(G_AHEAD):
            fire_gather(s, s)

        @pl.loop(0, n_steps, step=NBUF)
        def outer(g):
            for b in range(NBUF):
                s = g + b
                pltpu.make_async_copy(
                    lut_hbm.at[pl.ds(0, seq)], buf(b), gsem).wait()

                @plsc.parallel_loop(0, seq, unroll=8)
                def scale_row(i):
                    for c in range(D_MODEL // LANES):
                        sl = (b * seq + i, pl.ds(c * LANES, LANES))
                        rows_v[sl] = rows_v[sl] * SCALE

                pltpu.async_copy(buf(b), out_hbm.at[b0 + s], wsem)

                nb = (b + G_AHEAD) % NBUF

                @pl.when(jnp.logical_and(s >= NBUF - G_AHEAD,
                                         s + G_AHEAD < n_steps))
                def _():
                    pltpu.make_async_copy(
                        buf(nb), out_hbm.at[b0], wsem).wait()

                @pl.when(s + G_AHEAD < n_steps)
                def _():
                    fire_gather(s + G_AHEAD, nb)

        for _ in range(NBUF):
            pltpu.make_async_copy(buf(0), out_hbm.at[b0], wsem).wait()

    return emb_kernel


def kernel(x, lut):
    b, s = x.shape
    return _build(b, s)(x.astype(jnp.int32), lut)
